# baseline (speedup 1.0000x reference)
import jax
import jax.numpy as jnp
from jax.experimental import pallas as pl
from jax.experimental.pallas import tpu as pltpu

_N = 4096
_D = 256
_BI = 512
_BJ = 512


def _fused_body(adj_ref, x_ref, u_ref, out_ref, deg_ref):
    j = pl.program_id(1)
    nj = pl.num_programs(1)

    @pl.when(j == 0)
    def _init():
        out_ref[...] = jnp.zeros_like(out_ref)
        deg_ref[...] = jnp.zeros_like(deg_ref)

    a = adj_ref[...]
    m = (a > 0).astype(jnp.bfloat16)
    xb = x_ref[pl.ds(j * _BJ, _BJ), :].astype(jnp.bfloat16)
    out_ref[...] += jax.lax.dot_general(
        m, xb, (((0,), (0,)), ((), ())),
        preferred_element_type=jnp.float32)
    ones = jnp.ones((_BJ, 1), dtype=jnp.float32)
    deg_ref[...] += jax.lax.dot_general(
        a, ones, (((0,), (0,)), ((), ())),
        preferred_element_type=jnp.float32)

    @pl.when(j == nj - 1)
    def _epilogue():
        agg = out_ref[...] / deg_ref[...]
        h = jax.lax.dot_general(
            agg, u_ref[...], (((1,), (1,)), ((), ())),
            preferred_element_type=jnp.float32)
        out_ref[...] = jnp.maximum(h, 0.0)


def kernel(x, adj_mat, U):
    n, d = x.shape
    out = pl.pallas_call(
        _fused_body,
        grid=(n // _BI, n // _BJ),
        in_specs=[
            pl.BlockSpec((_BJ, _BI), lambda i, j: (j, i)),
            pl.BlockSpec((n, d), lambda i, j: (0, 0)),
            pl.BlockSpec((d, d), lambda i, j: (0, 0)),
        ],
        out_specs=pl.BlockSpec((_BI, d), lambda i, j: (i, 0)),
        out_shape=jax.ShapeDtypeStruct((n, d), jnp.float32),
        scratch_shapes=[pltpu.VMEM((_BI, 1), jnp.float32)],
        compiler_params=pltpu.CompilerParams(
            dimension_semantics=("parallel", "arbitrary")),
    )(adj_mat, x, U)
    return out[None, :, :]
